# baseline (speedup 1.0000x reference)
import functools

import jax
import jax.numpy as jnp
from jax import lax
from jax.experimental import pallas as pl
from jax.experimental.pallas import tpu as pltpu
from jax.experimental.pallas import tpu_sc as plsc

_NC = 2
_NS = 16
_K = 128


@functools.lru_cache(maxsize=None)
def _make_agg(N, E, D):
    NW = _NC * _NS
    EPW = E // NW
    assert EPW * NW == E
    KC = 80
    C = EPW // KC
    assert C * KC == EPW
    NB = 3
    ZR = KC
    RC = N // ZR
    assert RC * ZR == N
    mesh = plsc.VectorSubcoreMesh(core_axis_name="c", subcore_axis_name="s")

    @functools.partial(
        pl.kernel,
        out_type=jax.ShapeDtypeStruct((_NC, N, D), jnp.float32),
        mesh=mesh,
        scratch_types=[
            pltpu.VMEM((EPW,), jnp.int32),
            [pltpu.VMEM((KC,), jnp.int32) for _ in range(NB)],
            [pltpu.VMEM((KC, D), jnp.float32) for _ in range(NB)],
            pltpu.VMEM_SHARED((N, D), jnp.float32),
            pltpu.SemaphoreType.DMA,
            [pltpu.SemaphoreType.DMA for _ in range(NB)],
            [pltpu.SemaphoreType.DMA for _ in range(NB)],
        ],
    )
    def agg(x_hbm, ei_hbm, out_hbm, src_st, dstb, bufs, table,
            ssem, gsem, csem):
        cid = lax.axis_index("c")
        sid = lax.axis_index("s")
        wid = sid * _NC + cid
        w0 = wid * EPW

        pltpu.async_copy(ei_hbm.at[pl.ds(w0, EPW)], src_st, ssem)

        z16 = jnp.zeros((16,), jnp.float32)

        def zrow(r, carry):
            for j in range(D // 16):
                bufs[0][r, pl.ds(j * 16, 16)] = z16
            return carry

        lax.fori_loop(0, ZR, zrow, 0)
        n_rc = (RC - sid + _NS - 1) // _NS

        def start_gather(c, b):
            pltpu.async_copy(x_hbm.at[src_st.at[pl.ds(c * KC, KC)]],
                             bufs[b], gsem[b])
            pltpu.async_copy(ei_hbm.at[pl.ds(E + w0 + c * KC, KC)],
                             dstb[b], gsem[b])

        def wait_gather(b):
            pltpu.make_async_copy(x_hbm.at[pl.ds(0, KC)], bufs[b],
                                  gsem[b]).wait()
            pltpu.make_async_copy(ei_hbm.at[pl.ds(0, KC)], dstb[b],
                                  gsem[b]).wait()

        def wait_scatter(b):
            pltpu.make_async_copy(bufs[b], table.at[dstb[b]], csem[b]).wait()

        pltpu.make_async_copy(ei_hbm.at[pl.ds(0, EPW)], src_st, ssem).wait()
        for b in range(1, NB):
            if b < C:
                start_gather(b, b)

        def zbody(t, carry):
            pltpu.sync_copy(bufs[0].at[pl.ds(0, ZR)],
                            table.at[pl.ds((sid + t * _NS) * ZR, ZR)])
            return carry

        lax.fori_loop(0, n_rc, zbody, 0)
        start_gather(0, 0)
        plsc.subcore_barrier()

        def body(j, carry):
            for b in range(NB):
                c = NB * j + b
                cn = c + NB

                @pl.when(c < C)
                def _():
                    wait_gather(b)
                    pltpu.async_copy(bufs[b], table.at[dstb[b]], csem[b],
                                     add=True)

                @pl.when(cn < C)
                def _():
                    wait_scatter(b)
                    start_gather(cn, b)

            return carry

        lax.fori_loop(0, (C + NB - 1) // NB, body, 0)
        for b in range(NB):
            if b < C:
                wait_scatter(b)
        plsc.subcore_barrier()

        DR = (N // _NS) // 8 * 8
        pltpu.sync_copy(table.at[pl.ds(sid * DR, DR)],
                        out_hbm.at[cid, pl.ds(sid * DR, DR)])
        rem = N - _NS * DR
        if rem:
            @pl.when(sid == _NS - 1)
            def _():
                pltpu.sync_copy(table.at[pl.ds(_NS * DR, rem)],
                                out_hbm.at[cid, pl.ds(_NS * DR, rem)])

    return agg


def _layer_mlp_body(x_ref, ag_ref, W1_ref, b1_ref, g_ref, bt_ref, W2_ref,
                    b2_ref, o_ref):
    h = x_ref[...] + ag_ref[0] + ag_ref[1]
    h = jnp.dot(h, W1_ref[...], preferred_element_type=jnp.float32) + b1_ref[...]
    h = jnp.maximum(h * g_ref[...] + bt_ref[...], 0.0)
    h = jnp.dot(h, W2_ref[...], preferred_element_type=jnp.float32) + b2_ref[...]
    o_ref[...] = jnp.maximum(h, 0.0)


def _final_mlp_body(x_ref, ag_ref, W1_ref, b1_ref, g_ref, bt_ref, W2_ref,
                    b2_ref, cW1_ref, cb1_ref, cW2_ref, cb2_ref, o_ref):
    h = x_ref[...] + ag_ref[0] + ag_ref[1]
    h = jnp.dot(h, W1_ref[...], preferred_element_type=jnp.float32) + b1_ref[...]
    h = jnp.maximum(h * g_ref[...] + bt_ref[...], 0.0)
    h = jnp.dot(h, W2_ref[...], preferred_element_type=jnp.float32) + b2_ref[...]
    h = jnp.maximum(h, 0.0)
    h = jnp.maximum(
        jnp.dot(h, cW1_ref[...], preferred_element_type=jnp.float32) + cb1_ref[...],
        0.0)
    o_ref[...] = jnp.dot(h, cW2_ref[...], preferred_element_type=jnp.float32) + cb2_ref[...]


def _row_specs(N, D, BN):
    x_spec = pl.BlockSpec((BN, D), lambda i: (i, 0))
    ag_spec = pl.BlockSpec((_NC, BN, D), lambda i: (0, i, 0))
    return x_spec, ag_spec


def _w_spec(a, b):
    return pl.BlockSpec((a, b), lambda i: (0, 0))


def _mlp_layer(x, ag, W1, b1, g, bt, W2, b2):
    N, D = x.shape
    H = W1.shape[1]
    BN = 1000
    x_spec, ag_spec = _row_specs(N, D, BN)
    return pl.pallas_call(
        _layer_mlp_body,
        grid=(N // BN,),
        in_specs=[x_spec, ag_spec, _w_spec(D, H), _w_spec(1, H), _w_spec(1, H),
                  _w_spec(1, H), _w_spec(H, H), _w_spec(1, H)],
        out_specs=pl.BlockSpec((BN, H), lambda i: (i, 0)),
        out_shape=jax.ShapeDtypeStruct((N, H), jnp.float32),
    )(x, ag, W1, b1.reshape(1, H), g.reshape(1, H), bt.reshape(1, H), W2,
      b2.reshape(1, H))


def _mlp_final(x, ag, W1, b1, g, bt, W2, b2, cW1, cb1, cW2, cb2):
    N, D = x.shape
    H = W1.shape[1]
    O = cW2.shape[1]
    BN = 1000
    x_spec, ag_spec = _row_specs(N, D, BN)
    return pl.pallas_call(
        _final_mlp_body,
        grid=(N // BN,),
        in_specs=[x_spec, ag_spec, _w_spec(D, H), _w_spec(1, H), _w_spec(1, H),
                  _w_spec(1, H), _w_spec(H, H), _w_spec(1, H), _w_spec(H, H),
                  _w_spec(1, H), _w_spec(H, O), _w_spec(1, O)],
        out_specs=pl.BlockSpec((BN, O), lambda i: (i, 0)),
        out_shape=jax.ShapeDtypeStruct((N, O), jnp.float32),
    )(x, ag, W1, b1.reshape(1, H), g.reshape(1, H), bt.reshape(1, H), W2,
      b2.reshape(1, H), cW1, cb1.reshape(1, H), cW2, cb2.reshape(1, O))


def kernel(x, edge_index, l0_W1, l0_b1, l0_g, l0_bt, l0_W2, l0_b2,
           l1_W1, l1_b1, l1_g, l1_bt, l1_W2, l1_b2,
           c_W1, c_b1, c_W2, c_b2):
    N, D = x.shape
    E = edge_index.shape[1]
    ei = edge_index.reshape(2 * E)
    agg = _make_agg(N, E, D)
    ag0 = agg(x, ei)
    h = _mlp_layer(x, ag0, l0_W1, l0_b1, l0_g, l0_bt, l0_W2, l0_b2)
    ag1 = agg(h, ei)
    return _mlp_final(h, ag1, l1_W1, l1_b1, l1_g, l1_bt, l1_W2, l1_b2,
                      c_W1, c_b1, c_W2, c_b2)


# speedup vs baseline: 1.0074x; 1.0074x over previous
import functools

import jax
import jax.numpy as jnp
from jax import lax
from jax.experimental import pallas as pl
from jax.experimental.pallas import tpu as pltpu
from jax.experimental.pallas import tpu_sc as plsc

_NC = 2
_NS = 16
_K = 128


@functools.lru_cache(maxsize=None)
def _make_agg(N, E, D):
    NW = _NC * _NS
    EPW = E // NW
    assert EPW * NW == E
    KC = 80
    C = EPW // KC
    assert C * KC == EPW
    NB = 3
    ZR = KC
    RC = N // ZR
    assert RC * ZR == N
    mesh = plsc.VectorSubcoreMesh(core_axis_name="c", subcore_axis_name="s")

    @functools.partial(
        pl.kernel,
        out_type=jax.ShapeDtypeStruct((_NC, N, D), jnp.float32),
        mesh=mesh,
        scratch_types=[
            pltpu.VMEM((EPW,), jnp.int32),
            [pltpu.VMEM((KC,), jnp.int32) for _ in range(NB)],
            [pltpu.VMEM((KC, D), jnp.float32) for _ in range(NB)],
            pltpu.VMEM_SHARED((N, D), jnp.float32),
            pltpu.SemaphoreType.DMA,
            [pltpu.SemaphoreType.DMA for _ in range(NB)],
            [pltpu.SemaphoreType.DMA for _ in range(NB)],
        ],
    )
    def agg(x_hbm, ei_hbm, out_hbm, src_st, dstb, bufs, table,
            ssem, gsem, csem):
        cid = lax.axis_index("c")
        sid = lax.axis_index("s")
        wid = sid * _NC + cid
        w0 = wid * EPW

        pltpu.async_copy(ei_hbm.at[pl.ds(w0, EPW)], src_st, ssem)

        z16 = jnp.zeros((16,), jnp.float32)

        def zrow(r, carry):
            for j in range(D // 16):
                bufs[0][r, pl.ds(j * 16, 16)] = z16
            return carry

        lax.fori_loop(0, ZR, zrow, 0)
        n_rc = (RC - sid + _NS - 1) // _NS

        def start_gather(c, b):
            pltpu.async_copy(x_hbm.at[src_st.at[pl.ds(c * KC, KC)]],
                             bufs[b], gsem[b])
            pltpu.async_copy(ei_hbm.at[pl.ds(E + w0 + c * KC, KC)],
                             dstb[b], gsem[b])

        def wait_gather(b):
            pltpu.make_async_copy(x_hbm.at[pl.ds(0, KC)], bufs[b],
                                  gsem[b]).wait()
            pltpu.make_async_copy(ei_hbm.at[pl.ds(0, KC)], dstb[b],
                                  gsem[b]).wait()

        def wait_scatter(b):
            pltpu.make_async_copy(bufs[b], table.at[dstb[b]], csem[b]).wait()

        def zbody(t, carry):
            pltpu.sync_copy(bufs[0].at[pl.ds(0, ZR)],
                            table.at[pl.ds((sid + t * _NS) * ZR, ZR)])
            return carry

        lax.fori_loop(0, n_rc, zbody, 0)

        pltpu.make_async_copy(ei_hbm.at[pl.ds(0, EPW)], src_st, ssem).wait()
        for b in range(NB):
            if b < C:
                start_gather(b, b)
        plsc.subcore_barrier()

        def body(j, carry):
            for b in range(NB):
                c = NB * j + b
                cn = c + NB

                @pl.when(c < C)
                def _():
                    wait_gather(b)
                    pltpu.async_copy(bufs[b], table.at[dstb[b]], csem[b],
                                     add=True)

                @pl.when(cn < C)
                def _():
                    wait_scatter(b)
                    start_gather(cn, b)

            return carry

        lax.fori_loop(0, (C + NB - 1) // NB, body, 0)
        for b in range(NB):
            if b < C:
                wait_scatter(b)
        plsc.subcore_barrier()

        def dbody(t, carry):
            r0 = (sid + t * _NS) * ZR
            pltpu.sync_copy(table.at[pl.ds(r0, ZR)],
                            out_hbm.at[cid, pl.ds(r0, ZR)])
            return carry

        lax.fori_loop(0, n_rc, dbody, 0)

    return agg


def _layer_mlp_body(x_ref, ag_ref, W1_ref, b1_ref, g_ref, bt_ref, W2_ref,
                    b2_ref, o_ref):
    h = x_ref[...] + ag_ref[0] + ag_ref[1]
    h = jnp.dot(h, W1_ref[...], preferred_element_type=jnp.float32) + b1_ref[...]
    h = jnp.maximum(h * g_ref[...] + bt_ref[...], 0.0)
    h = jnp.dot(h, W2_ref[...], preferred_element_type=jnp.float32) + b2_ref[...]
    o_ref[...] = jnp.maximum(h, 0.0)


def _final_mlp_body(x_ref, ag_ref, W1_ref, b1_ref, g_ref, bt_ref, W2_ref,
                    b2_ref, cW1_ref, cb1_ref, cW2_ref, cb2_ref, o_ref):
    h = x_ref[...] + ag_ref[0] + ag_ref[1]
    h = jnp.dot(h, W1_ref[...], preferred_element_type=jnp.float32) + b1_ref[...]
    h = jnp.maximum(h * g_ref[...] + bt_ref[...], 0.0)
    h = jnp.dot(h, W2_ref[...], preferred_element_type=jnp.float32) + b2_ref[...]
    h = jnp.maximum(h, 0.0)
    h = jnp.maximum(
        jnp.dot(h, cW1_ref[...], preferred_element_type=jnp.float32) + cb1_ref[...],
        0.0)
    o_ref[...] = jnp.dot(h, cW2_ref[...], preferred_element_type=jnp.float32) + cb2_ref[...]


def _row_specs(N, D, BN):
    x_spec = pl.BlockSpec((BN, D), lambda i: (i, 0))
    ag_spec = pl.BlockSpec((_NC, BN, D), lambda i: (0, i, 0))
    return x_spec, ag_spec


def _w_spec(a, b):
    return pl.BlockSpec((a, b), lambda i: (0, 0))


def _mlp_layer(x, ag, W1, b1, g, bt, W2, b2):
    N, D = x.shape
    H = W1.shape[1]
    BN = 1000
    x_spec, ag_spec = _row_specs(N, D, BN)
    return pl.pallas_call(
        _layer_mlp_body,
        grid=(N // BN,),
        in_specs=[x_spec, ag_spec, _w_spec(D, H), _w_spec(1, H), _w_spec(1, H),
                  _w_spec(1, H), _w_spec(H, H), _w_spec(1, H)],
        out_specs=pl.BlockSpec((BN, H), lambda i: (i, 0)),
        out_shape=jax.ShapeDtypeStruct((N, H), jnp.float32),
    )(x, ag, W1, b1.reshape(1, H), g.reshape(1, H), bt.reshape(1, H), W2,
      b2.reshape(1, H))


def _mlp_final(x, ag, W1, b1, g, bt, W2, b2, cW1, cb1, cW2, cb2):
    N, D = x.shape
    H = W1.shape[1]
    O = cW2.shape[1]
    BN = 1000
    x_spec, ag_spec = _row_specs(N, D, BN)
    return pl.pallas_call(
        _final_mlp_body,
        grid=(N // BN,),
        in_specs=[x_spec, ag_spec, _w_spec(D, H), _w_spec(1, H), _w_spec(1, H),
                  _w_spec(1, H), _w_spec(H, H), _w_spec(1, H), _w_spec(H, H),
                  _w_spec(1, H), _w_spec(H, O), _w_spec(1, O)],
        out_specs=pl.BlockSpec((BN, O), lambda i: (i, 0)),
        out_shape=jax.ShapeDtypeStruct((N, O), jnp.float32),
    )(x, ag, W1, b1.reshape(1, H), g.reshape(1, H), bt.reshape(1, H), W2,
      b2.reshape(1, H), cW1, cb1.reshape(1, H), cW2, cb2.reshape(1, O))


def kernel(x, edge_index, l0_W1, l0_b1, l0_g, l0_bt, l0_W2, l0_b2,
           l1_W1, l1_b1, l1_g, l1_bt, l1_W2, l1_b2,
           c_W1, c_b1, c_W2, c_b2):
    N, D = x.shape
    E = edge_index.shape[1]
    ei = edge_index.reshape(2 * E)
    agg = _make_agg(N, E, D)
    ag0 = agg(x, ei)
    h = _mlp_layer(x, ag0, l0_W1, l0_b1, l0_g, l0_bt, l0_W2, l0_b2)
    ag1 = agg(h, ei)
    return _mlp_final(h, ag1, l1_W1, l1_b1, l1_g, l1_bt, l1_W2, l1_b2,
                      c_W1, c_b1, c_W2, c_b2)


# speedup vs baseline: 1.0386x; 1.0310x over previous
import functools

import jax
import jax.numpy as jnp
from jax import lax
from jax.experimental import pallas as pl
from jax.experimental.pallas import tpu as pltpu
from jax.experimental.pallas import tpu_sc as plsc

_NC = 2
_NS = 16
_K = 128


@functools.lru_cache(maxsize=None)
def _make_agg(N, E, D):
    NW = _NC * _NS
    EPW = E // NW
    assert EPW * NW == E
    KC = 80
    C = EPW // KC
    assert C * KC == EPW
    NB = 3
    ZR = KC
    RC = N // ZR
    assert RC * ZR == N
    mesh = plsc.VectorSubcoreMesh(core_axis_name="c", subcore_axis_name="s")

    @functools.partial(
        pl.kernel,
        out_type=jax.ShapeDtypeStruct((_NC, N, D), jnp.float32),
        mesh=mesh,
        scratch_types=[
            pltpu.VMEM((EPW,), jnp.int32),
            [pltpu.VMEM((KC,), jnp.int32) for _ in range(NB)],
            [pltpu.VMEM((KC, D), jnp.float32) for _ in range(NB)],
            pltpu.VMEM_SHARED((N, D), jnp.float32),
            pltpu.SemaphoreType.DMA,
            [pltpu.SemaphoreType.DMA for _ in range(NB)],
            [pltpu.SemaphoreType.DMA for _ in range(NB)],
        ],
    )
    def agg(x_hbm, ei_hbm, out_hbm, src_st, dstb, bufs, table,
            ssem, gsem, csem):
        cid = lax.axis_index("c")
        sid = lax.axis_index("s")
        wid = sid * _NC + cid
        w0 = wid * EPW

        pltpu.async_copy(ei_hbm.at[pl.ds(w0, EPW)], src_st, ssem)

        z16 = jnp.zeros((16,), jnp.float32)

        def zrow(r, carry):
            for j in range(D // 16):
                bufs[0][r, pl.ds(j * 16, 16)] = z16
            return carry

        lax.fori_loop(0, ZR, zrow, 0)
        n_rc = (RC - sid + _NS - 1) // _NS

        def start_gather(c, b):
            pltpu.async_copy(x_hbm.at[src_st.at[pl.ds(c * KC, KC)]],
                             bufs[b], gsem[b])
            pltpu.async_copy(ei_hbm.at[pl.ds(E + w0 + c * KC, KC)],
                             dstb[b], gsem[b])

        def wait_gather(b):
            pltpu.make_async_copy(x_hbm.at[pl.ds(0, KC)], bufs[b],
                                  gsem[b]).wait()
            pltpu.make_async_copy(ei_hbm.at[pl.ds(0, KC)], dstb[b],
                                  gsem[b]).wait()

        def wait_scatter(b):
            pltpu.make_async_copy(bufs[b], table.at[dstb[b]], csem[b]).wait()

        def zbody(t, carry):
            pltpu.sync_copy(bufs[0].at[pl.ds(0, ZR)],
                            table.at[pl.ds((sid + t * _NS) * ZR, ZR)])
            return carry

        lax.fori_loop(0, n_rc, zbody, 0)

        pltpu.make_async_copy(ei_hbm.at[pl.ds(0, EPW)], src_st, ssem).wait()
        for b in range(NB):
            if b < C:
                start_gather(b, b)
        plsc.subcore_barrier()

        def body(j, carry):
            for b in range(NB):
                c = NB * j + b
                cn = c + NB

                @pl.when(c < C)
                def _():
                    wait_gather(b)
                    pltpu.async_copy(bufs[b], table.at[dstb[b]], csem[b],
                                     add=True)

                @pl.when(cn < C)
                def _():
                    wait_scatter(b)
                    start_gather(cn, b)

            return carry

        lax.fori_loop(0, (C + NB - 1) // NB, body, 0)
        for b in range(NB):
            if b < C:
                wait_scatter(b)
        plsc.subcore_barrier()

        def dbody(t, carry):
            r0 = (sid + t * _NS) * ZR
            pltpu.sync_copy(table.at[pl.ds(r0, ZR)],
                            out_hbm.at[cid, pl.ds(r0, ZR)])
            return carry

        lax.fori_loop(0, n_rc, dbody, 0)

    return agg


def _layer_mlp_body(x_ref, ag_ref, W1_ref, b1_ref, g_ref, bt_ref, W2_ref,
                    b2_ref, o_ref):
    h = x_ref[...] + ag_ref[0] + ag_ref[1]
    h = jnp.dot(h, W1_ref[...], preferred_element_type=jnp.float32) + b1_ref[...]
    h = jnp.maximum(h * g_ref[...] + bt_ref[...], 0.0)
    h = jnp.dot(h, W2_ref[...], preferred_element_type=jnp.float32) + b2_ref[...]
    o_ref[...] = jnp.maximum(h, 0.0)


def _final_mlp_body(x_ref, ag_ref, W1_ref, b1_ref, g_ref, bt_ref, W2_ref,
                    b2_ref, cW1_ref, cb1_ref, cW2_ref, cb2_ref, o_ref):
    h = x_ref[...] + ag_ref[0] + ag_ref[1]
    h = jnp.dot(h, W1_ref[...], preferred_element_type=jnp.float32) + b1_ref[...]
    h = jnp.maximum(h * g_ref[...] + bt_ref[...], 0.0)
    h = jnp.dot(h, W2_ref[...], preferred_element_type=jnp.float32) + b2_ref[...]
    h = jnp.maximum(h, 0.0)
    h = jnp.maximum(
        jnp.dot(h, cW1_ref[...], preferred_element_type=jnp.float32) + cb1_ref[...],
        0.0)
    o_ref[...] = jnp.dot(h, cW2_ref[...], preferred_element_type=jnp.float32) + cb2_ref[...]


def _row_specs(N, D, BN):
    x_spec = pl.BlockSpec((BN, D), lambda i: (i, 0))
    ag_spec = pl.BlockSpec((_NC, BN, D), lambda i: (0, i, 0))
    return x_spec, ag_spec


def _w_spec(a, b):
    return pl.BlockSpec((a, b), lambda i: (0, 0))


def _mlp_layer(x, ag, W1, b1, g, bt, W2, b2):
    N, D = x.shape
    H = W1.shape[1]
    BN = 2000
    x_spec, ag_spec = _row_specs(N, D, BN)
    return pl.pallas_call(
        _layer_mlp_body,
        grid=(N // BN,),
        in_specs=[x_spec, ag_spec, _w_spec(D, H), _w_spec(1, H), _w_spec(1, H),
                  _w_spec(1, H), _w_spec(H, H), _w_spec(1, H)],
        out_specs=pl.BlockSpec((BN, H), lambda i: (i, 0)),
        out_shape=jax.ShapeDtypeStruct((N, H), jnp.float32),
    )(x, ag, W1, b1.reshape(1, H), g.reshape(1, H), bt.reshape(1, H), W2,
      b2.reshape(1, H))


def _mlp_final(x, ag, W1, b1, g, bt, W2, b2, cW1, cb1, cW2, cb2):
    N, D = x.shape
    H = W1.shape[1]
    O = cW2.shape[1]
    BN = 2000
    x_spec, ag_spec = _row_specs(N, D, BN)
    return pl.pallas_call(
        _final_mlp_body,
        grid=(N // BN,),
        in_specs=[x_spec, ag_spec, _w_spec(D, H), _w_spec(1, H), _w_spec(1, H),
                  _w_spec(1, H), _w_spec(H, H), _w_spec(1, H), _w_spec(H, H),
                  _w_spec(1, H), _w_spec(H, O), _w_spec(1, O)],
        out_specs=pl.BlockSpec((BN, O), lambda i: (i, 0)),
        out_shape=jax.ShapeDtypeStruct((N, O), jnp.float32),
    )(x, ag, W1, b1.reshape(1, H), g.reshape(1, H), bt.reshape(1, H), W2,
      b2.reshape(1, H), cW1, cb1.reshape(1, H), cW2, cb2.reshape(1, O))


def kernel(x, edge_index, l0_W1, l0_b1, l0_g, l0_bt, l0_W2, l0_b2,
           l1_W1, l1_b1, l1_g, l1_bt, l1_W2, l1_b2,
           c_W1, c_b1, c_W2, c_b2):
    N, D = x.shape
    E = edge_index.shape[1]
    ei = edge_index.reshape(2 * E)
    agg = _make_agg(N, E, D)
    ag0 = agg(x, ei)
    h = _mlp_layer(x, ag0, l0_W1, l0_b1, l0_g, l0_bt, l0_W2, l0_b2)
    ag1 = agg(h, ei)
    return _mlp_final(h, ag1, l1_W1, l1_b1, l1_g, l1_bt, l1_W2, l1_b2,
                      c_W1, c_b1, c_W2, c_b2)


# speedup vs baseline: 1.0433x; 1.0046x over previous
import functools

import jax
import jax.numpy as jnp
from jax import lax
from jax.experimental import pallas as pl
from jax.experimental.pallas import tpu as pltpu
from jax.experimental.pallas import tpu_sc as plsc

_NC = 2
_NS = 16
_K = 128


@functools.lru_cache(maxsize=None)
def _make_agg(N, E, D):
    NW = _NC * _NS
    EPW = E // NW
    assert EPW * NW == E
    KC = 80
    C = EPW // KC
    assert C * KC == EPW
    NB = 3
    ZR = KC
    RC = N // ZR
    assert RC * ZR == N
    mesh = plsc.VectorSubcoreMesh(core_axis_name="c", subcore_axis_name="s")

    @functools.partial(
        pl.kernel,
        out_type=jax.ShapeDtypeStruct((_NC, N, D), jnp.float32),
        mesh=mesh,
        scratch_types=[
            pltpu.VMEM((EPW,), jnp.int32),
            [pltpu.VMEM((KC,), jnp.int32) for _ in range(NB)],
            [pltpu.VMEM((KC, D), jnp.float32) for _ in range(NB)],
            pltpu.VMEM_SHARED((N, D), jnp.float32),
            pltpu.SemaphoreType.DMA,
            [pltpu.SemaphoreType.DMA for _ in range(NB)],
            [pltpu.SemaphoreType.DMA for _ in range(NB)],
        ],
    )
    def agg(x_hbm, ei_hbm, out_hbm, src_st, dstb, bufs, table,
            ssem, gsem, csem):
        cid = lax.axis_index("c")
        sid = lax.axis_index("s")
        wid = sid * _NC + cid
        w0 = wid * EPW

        pltpu.async_copy(ei_hbm.at[pl.ds(w0, EPW)], src_st, ssem)

        z16 = jnp.zeros((16,), jnp.float32)

        def zrow(r, carry):
            for j in range(D // 16):
                bufs[0][r, pl.ds(j * 16, 16)] = z16
            return carry

        lax.fori_loop(0, ZR, zrow, 0)
        n_rc = (RC - sid + _NS - 1) // _NS

        def start_gather(c, b):
            pltpu.async_copy(x_hbm.at[src_st.at[pl.ds(c * KC, KC)]],
                             bufs[b], gsem[b])
            pltpu.async_copy(ei_hbm.at[pl.ds(E + w0 + c * KC, KC)],
                             dstb[b], gsem[b])

        def wait_gather(b):
            pltpu.make_async_copy(x_hbm.at[pl.ds(0, KC)], bufs[b],
                                  gsem[b]).wait()
            pltpu.make_async_copy(ei_hbm.at[pl.ds(0, KC)], dstb[b],
                                  gsem[b]).wait()

        def wait_scatter(b):
            pltpu.make_async_copy(bufs[b], table.at[dstb[b]], csem[b]).wait()

        def zbody(t, carry):
            pltpu.sync_copy(bufs[0].at[pl.ds(0, ZR)],
                            table.at[pl.ds((sid + t * _NS) * ZR, ZR)])
            return carry

        lax.fori_loop(0, n_rc, zbody, 0)

        pltpu.make_async_copy(ei_hbm.at[pl.ds(0, EPW)], src_st, ssem).wait()
        for b in range(NB):
            if b < C:
                start_gather(b, b)
        plsc.subcore_barrier()

        def body(j, carry):
            for b in range(NB):
                c = NB * j + b
                cn = c + NB

                @pl.when(c < C)
                def _():
                    wait_gather(b)
                    pltpu.async_copy(bufs[b], table.at[dstb[b]], csem[b],
                                     add=True)

                @pl.when(cn < C)
                def _():
                    wait_scatter(b)
                    start_gather(cn, b)

            return carry

        lax.fori_loop(0, (C + NB - 1) // NB, body, 0)
        for b in range(NB):
            if b < C:
                wait_scatter(b)
        plsc.subcore_barrier()

        def dbody(t, carry):
            r0 = (sid + t * _NS) * ZR
            pltpu.sync_copy(table.at[pl.ds(r0, ZR)],
                            out_hbm.at[cid, pl.ds(r0, ZR)])
            return carry

        lax.fori_loop(0, n_rc, dbody, 0)

    return agg


def _layer_mlp_body(x_ref, ag_ref, W1_ref, b1_ref, g_ref, bt_ref, W2_ref,
                    b2_ref, o_ref):
    h = x_ref[...] + ag_ref[0] + ag_ref[1]
    h = jnp.dot(h, W1_ref[...], preferred_element_type=jnp.float32) + b1_ref[...]
    h = jnp.maximum(h * g_ref[...] + bt_ref[...], 0.0)
    h = jnp.dot(h, W2_ref[...], preferred_element_type=jnp.float32) + b2_ref[...]
    o_ref[...] = jnp.maximum(h, 0.0)


def _final_mlp_body(x_ref, ag_ref, W1_ref, b1_ref, g_ref, bt_ref, W2_ref,
                    b2_ref, cW1_ref, cb1_ref, cW2_ref, cb2_ref, o_ref):
    h = x_ref[...] + ag_ref[0] + ag_ref[1]
    h = jnp.dot(h, W1_ref[...], preferred_element_type=jnp.float32) + b1_ref[...]
    h = jnp.maximum(h * g_ref[...] + bt_ref[...], 0.0)
    h = jnp.dot(h, W2_ref[...], preferred_element_type=jnp.float32) + b2_ref[...]
    h = jnp.maximum(h, 0.0)
    h = jnp.maximum(
        jnp.dot(h, cW1_ref[...], preferred_element_type=jnp.float32) + cb1_ref[...],
        0.0)
    o_ref[...] = jnp.dot(h, cW2_ref[...], preferred_element_type=jnp.float32) + cb2_ref[...]


def _row_specs(N, D, BN):
    x_spec = pl.BlockSpec((BN, D), lambda i: (i, 0))
    ag_spec = pl.BlockSpec((_NC, BN, D), lambda i: (0, i, 0))
    return x_spec, ag_spec


def _w_spec(a, b):
    return pl.BlockSpec((a, b), lambda i: (0, 0))


def _mlp_layer(x, ag, W1, b1, g, bt, W2, b2):
    N, D = x.shape
    H = W1.shape[1]
    BN = 5000
    x_spec, ag_spec = _row_specs(N, D, BN)
    return pl.pallas_call(
        _layer_mlp_body,
        grid=(N // BN,),
        in_specs=[x_spec, ag_spec, _w_spec(D, H), _w_spec(1, H), _w_spec(1, H),
                  _w_spec(1, H), _w_spec(H, H), _w_spec(1, H)],
        out_specs=pl.BlockSpec((BN, H), lambda i: (i, 0)),
        out_shape=jax.ShapeDtypeStruct((N, H), jnp.float32),
    )(x, ag, W1, b1.reshape(1, H), g.reshape(1, H), bt.reshape(1, H), W2,
      b2.reshape(1, H))


def _mlp_final(x, ag, W1, b1, g, bt, W2, b2, cW1, cb1, cW2, cb2):
    N, D = x.shape
    H = W1.shape[1]
    O = cW2.shape[1]
    BN = 5000
    x_spec, ag_spec = _row_specs(N, D, BN)
    return pl.pallas_call(
        _final_mlp_body,
        grid=(N // BN,),
        in_specs=[x_spec, ag_spec, _w_spec(D, H), _w_spec(1, H), _w_spec(1, H),
                  _w_spec(1, H), _w_spec(H, H), _w_spec(1, H), _w_spec(H, H),
                  _w_spec(1, H), _w_spec(H, O), _w_spec(1, O)],
        out_specs=pl.BlockSpec((BN, O), lambda i: (i, 0)),
        out_shape=jax.ShapeDtypeStruct((N, O), jnp.float32),
    )(x, ag, W1, b1.reshape(1, H), g.reshape(1, H), bt.reshape(1, H), W2,
      b2.reshape(1, H), cW1, cb1.reshape(1, H), cW2, cb2.reshape(1, O))


def kernel(x, edge_index, l0_W1, l0_b1, l0_g, l0_bt, l0_W2, l0_b2,
           l1_W1, l1_b1, l1_g, l1_bt, l1_W2, l1_b2,
           c_W1, c_b1, c_W2, c_b2):
    N, D = x.shape
    E = edge_index.shape[1]
    ei = edge_index.reshape(2 * E)
    agg = _make_agg(N, E, D)
    ag0 = agg(x, ei)
    h = _mlp_layer(x, ag0, l0_W1, l0_b1, l0_g, l0_bt, l0_W2, l0_b2)
    ag1 = agg(h, ei)
    return _mlp_final(h, ag1, l1_W1, l1_b1, l1_g, l1_bt, l1_W2, l1_b2,
                      c_W1, c_b1, c_W2, c_b2)
